# Initial kernel scaffold; baseline (speedup 1.0000x reference)
import functools

import jax
import jax.numpy as jnp
from jax import lax
from jax.experimental import pallas as pl
from jax.experimental.pallas import tpu as pltpu
from jax.experimental.pallas import tpu_sc as plsc

N = 10000
E = 320000
D_IN, HID, OUT = 128, 64, 128

NC, NS, LANES = 2, 16, 16
NW = NC * NS
EPW = E // NW
BLK = 80
NBLK = EPW // BLK
RPS = N // NS
CNT_W = 16
ZR = 125

TC_BLK = 1000


def _seg_body(with_counts, *refs):
    if with_counts:
        (vals, edges, out, cnt_out,
         src_v, dst_v, rows_v, ones_v, zero_v, zero_c, acc_sh, cnt_sh, sem) = refs
    else:
        (vals, edges, out,
         src_v, dst_v, rows_v, zero_v, acc_sh, sem) = refs
    c = lax.axis_index("c")
    s = lax.axis_index("s")
    wid = s * NC + c

    zvec = jnp.zeros((LANES,), jnp.float32)

    @pl.loop(0, ZR)
    def _(i):
        for j in range(0, HID, LANES):
            zero_v[i, pl.ds(j, LANES)] = zvec

    if with_counts:
        ovec = jnp.ones((LANES,), jnp.float32)

        @pl.loop(0, ZR)
        def _(i):
            zero_c[i, :] = zvec

        @pl.loop(0, BLK)
        def _(i):
            ones_v[i, :] = ovec

    for k in range(RPS // ZR):
        r0 = s * RPS + k * ZR
        pltpu.sync_copy(zero_v, acc_sh.at[pl.ds(r0, ZR)])
        if with_counts:
            pltpu.sync_copy(zero_c, cnt_sh.at[pl.ds(r0, ZR)])
    plsc.subcore_barrier()

    base_w = wid * EPW

    @pl.loop(0, NBLK)
    def _(i):
        base = base_w + i * BLK
        pltpu.sync_copy(edges.at[0, pl.ds(base, BLK)], src_v)
        pltpu.sync_copy(edges.at[1, pl.ds(base, BLK)], dst_v)
        pltpu.async_copy(vals.at[src_v], rows_v, sem).wait()
        pltpu.sync_copy(rows_v, acc_sh.at[dst_v], add=True)
        if with_counts:
            pltpu.sync_copy(ones_v, cnt_sh.at[dst_v], add=True)

    plsc.subcore_barrier()

    r0 = s * RPS
    pltpu.sync_copy(acc_sh.at[pl.ds(r0, RPS)], out.at[c].at[pl.ds(r0, RPS)])
    if with_counts:
        pltpu.sync_copy(cnt_sh.at[pl.ds(r0, RPS)], cnt_out.at[c].at[pl.ds(r0, RPS)])


def _make_seg(with_counts):
    mesh = plsc.VectorSubcoreMesh(core_axis_name="c", subcore_axis_name="s")
    out_type = [jax.ShapeDtypeStruct((NC, N, HID), jnp.float32)]
    scratch = [
        pltpu.VMEM((BLK,), jnp.int32),
        pltpu.VMEM((BLK,), jnp.int32),
        pltpu.VMEM((BLK, HID), jnp.float32),
        pltpu.VMEM((ZR, HID), jnp.float32),
        pltpu.VMEM_SHARED((N, HID), jnp.float32),
        pltpu.SemaphoreType.DMA,
    ]
    if with_counts:
        out_type.append(jax.ShapeDtypeStruct((NC, N, CNT_W), jnp.float32))
        scratch.insert(3, pltpu.VMEM((BLK, CNT_W), jnp.float32))
        scratch.insert(5, pltpu.VMEM((ZR, CNT_W), jnp.float32))
        scratch.insert(7, pltpu.VMEM_SHARED((N, CNT_W), jnp.float32))
    return pl.kernel(
        functools.partial(_seg_body, with_counts),
        out_type=out_type if with_counts else out_type[0],
        mesh=mesh,
        scratch_types=scratch,
    )


_seg_with_counts = _make_seg(True)
_seg_plain = _make_seg(False)


def _tc1_body(x_ref, w_ref, b_ref, y_ref, xr_ref):
    prod = jnp.dot(x_ref[...], w_ref[...], preferred_element_type=jnp.float32)
    y_ref[...] = prod[:, :HID]
    xr_ref[...] = prod[:, HID:] + b_ref[...]


def _tc2_body(s0_ref, s1_ref, c0_ref, c1_ref, xr_ref, h_ref, inv_ref):
    inv = 1.0 / jnp.maximum(c0_ref[...] + c1_ref[...], 1.0)
    aggr = (s0_ref[...] + s1_ref[...]) * inv[:, 0:1]
    h_ref[...] = jnp.maximum(aggr + xr_ref[...], 0.0)
    inv_ref[...] = inv


def _tc3_body(s0_ref, s1_ref, inv_ref, h_ref, w_ref, b_ref, out_ref):
    aggr = (s0_ref[...] + s1_ref[...]) * inv_ref[:, 0:1]
    z = jnp.concatenate([aggr, h_ref[...]], axis=1)
    out_ref[...] = jnp.dot(z, w_ref[...], preferred_element_type=jnp.float32) + b_ref[...]


def _row_spec(width):
    return pl.BlockSpec((TC_BLK, width), lambda i: (i, 0))


def _full_spec(shape):
    return pl.BlockSpec(shape, lambda i: tuple(0 for _ in shape))


_GRID = N // TC_BLK

_tc1 = pl.pallas_call(
    _tc1_body,
    grid=(_GRID,),
    in_specs=[_row_spec(D_IN), _full_spec((D_IN, 2 * HID)), _full_spec((1, HID))],
    out_specs=[_row_spec(HID), _row_spec(HID)],
    out_shape=[jax.ShapeDtypeStruct((N, HID), jnp.float32)] * 2,
)

_tc2 = pl.pallas_call(
    _tc2_body,
    grid=(_GRID,),
    in_specs=[_row_spec(HID), _row_spec(HID), _row_spec(CNT_W), _row_spec(CNT_W),
              _row_spec(HID)],
    out_specs=[_row_spec(HID), _row_spec(CNT_W)],
    out_shape=[jax.ShapeDtypeStruct((N, HID), jnp.float32),
               jax.ShapeDtypeStruct((N, CNT_W), jnp.float32)],
)

_tc3 = pl.pallas_call(
    _tc3_body,
    grid=(_GRID,),
    in_specs=[_row_spec(HID), _row_spec(HID), _row_spec(CNT_W), _row_spec(HID),
              _full_spec((2 * HID, OUT)), _full_spec((1, OUT))],
    out_specs=_row_spec(OUT),
    out_shape=jax.ShapeDtypeStruct((N, OUT), jnp.float32),
)


def kernel(x, edge_index, W1_l, b1, W1_r, W2_l, b2, W2_r):
    w1 = jnp.concatenate([W1_l.T, W1_r.T], axis=1)
    w2 = jnp.concatenate([W2_l.T, W2_r.T], axis=0)
    y1, xr1 = _tc1(x, w1, b1[None, :])
    s1p, cntp = _seg_with_counts(y1, edge_index)
    h, inv = _tc2(s1p[0], s1p[1], cntp[0], cntp[1], xr1)
    s2p = _seg_plain(h, edge_index)
    out = _tc3(s2p[0], s2p[1], inv, h, w2, b2[None, :])
    return out


# speedup vs baseline: 5.9919x; 5.9919x over previous
import functools

import jax
import jax.numpy as jnp
from jax import lax
from jax.experimental import pallas as pl
from jax.experimental.pallas import tpu as pltpu
from jax.experimental.pallas import tpu_sc as plsc

N = 10000
E = 320000
D_IN, HID, OUT = 128, 64, 128

NC, NS, LANES = 2, 16, 16
NW = NC * NS
EPW = E // NW
BLK = 80
NBLK = EPW // BLK
N_PAD = 10240
RPS = N_PAD // NS
CNT_W = 16
ZR = 128

TC_BLK = 1000


def _seg_body(with_counts, *refs):
    if with_counts:
        (vals, edges, out, cnt_out,
         src_v, dst_v, rows_v, ones_v, zero_v, zero_c, acc_sh, cnt_sh, sem) = refs
    else:
        (vals, edges, out,
         src_v, dst_v, rows_v, zero_v, acc_sh, sem) = refs
    c = lax.axis_index("c")
    s = lax.axis_index("s")
    wid = s * NC + c

    zvec = jnp.zeros((LANES,), jnp.float32)

    @pl.loop(0, ZR)
    def _(i):
        for j in range(0, HID, LANES):
            zero_v[i, pl.ds(j, LANES)] = zvec

    if with_counts:
        ovec = jnp.ones((LANES,), jnp.float32)

        @pl.loop(0, ZR)
        def _(i):
            zero_c[i, :] = zvec

        @pl.loop(0, BLK)
        def _(i):
            ones_v[i, :] = ovec

    for k in range(RPS // ZR):
        r0 = s * RPS + k * ZR
        pltpu.sync_copy(zero_v, acc_sh.at[pl.ds(r0, ZR)])
        if with_counts:
            pltpu.sync_copy(zero_c, cnt_sh.at[pl.ds(r0, ZR)])
    plsc.subcore_barrier()

    base_w = wid * EPW

    @pl.loop(0, NBLK)
    def _(i):
        base = base_w + i * BLK
        pltpu.sync_copy(edges.at[pl.ds(base, BLK)], src_v)
        pltpu.sync_copy(edges.at[pl.ds(E + base, BLK)], dst_v)
        pltpu.async_copy(vals.at[src_v], rows_v, sem).wait()
        pltpu.sync_copy(rows_v, acc_sh.at[dst_v], add=True)
        if with_counts:
            pltpu.sync_copy(ones_v, cnt_sh.at[dst_v], add=True)

    plsc.subcore_barrier()

    r0 = s * RPS
    pltpu.sync_copy(acc_sh.at[pl.ds(r0, RPS)], out.at[c].at[pl.ds(r0, RPS)])
    if with_counts:
        pltpu.sync_copy(cnt_sh.at[pl.ds(r0, RPS)], cnt_out.at[c].at[pl.ds(r0, RPS)])


def _make_seg(with_counts):
    mesh = plsc.VectorSubcoreMesh(core_axis_name="c", subcore_axis_name="s")
    out_type = [jax.ShapeDtypeStruct((NC, N_PAD, HID), jnp.float32)]
    scratch = [
        pltpu.VMEM((BLK,), jnp.int32),
        pltpu.VMEM((BLK,), jnp.int32),
        pltpu.VMEM((BLK, HID), jnp.float32),
        pltpu.VMEM((ZR, HID), jnp.float32),
        pltpu.VMEM_SHARED((N_PAD, HID), jnp.float32),
        pltpu.SemaphoreType.DMA,
    ]
    if with_counts:
        out_type.append(jax.ShapeDtypeStruct((NC, N_PAD, CNT_W), jnp.float32))
        scratch.insert(3, pltpu.VMEM((BLK, CNT_W), jnp.float32))
        scratch.insert(5, pltpu.VMEM((ZR, CNT_W), jnp.float32))
        scratch.insert(7, pltpu.VMEM_SHARED((N_PAD, CNT_W), jnp.float32))
    return pl.kernel(
        functools.partial(_seg_body, with_counts),
        out_type=out_type if with_counts else out_type[0],
        mesh=mesh,
        scratch_types=scratch,
        compiler_params=pltpu.CompilerParams(use_tc_tiling_on_sc=False),
    )


_seg_with_counts = _make_seg(True)
_seg_plain = _make_seg(False)


def _tc1_body(x_ref, w_ref, b_ref, y_ref, xr_ref):
    prod = jnp.dot(x_ref[...], w_ref[...], preferred_element_type=jnp.float32)
    y_ref[...] = prod[:, :HID]
    xr_ref[...] = prod[:, HID:] + b_ref[...]


def _tc2_body(s0_ref, s1_ref, c0_ref, c1_ref, xr_ref, h_ref, inv_ref):
    inv = 1.0 / jnp.maximum(c0_ref[...] + c1_ref[...], 1.0)
    aggr = (s0_ref[...] + s1_ref[...]) * inv[:, 0:1]
    h_ref[...] = jnp.maximum(aggr + xr_ref[...], 0.0)
    inv_ref[...] = inv


def _tc3_body(s0_ref, s1_ref, inv_ref, h_ref, w_ref, b_ref, out_ref):
    aggr = (s0_ref[...] + s1_ref[...]) * inv_ref[:, 0:1]
    z = jnp.concatenate([aggr, h_ref[...]], axis=1)
    out_ref[...] = jnp.dot(z, w_ref[...], preferred_element_type=jnp.float32) + b_ref[...]


def _row_spec(width):
    return pl.BlockSpec((TC_BLK, width), lambda i: (i, 0))


def _full_spec(shape):
    return pl.BlockSpec(shape, lambda i: tuple(0 for _ in shape))


_GRID = N // TC_BLK

_tc1 = pl.pallas_call(
    _tc1_body,
    grid=(_GRID,),
    in_specs=[_row_spec(D_IN), _full_spec((D_IN, 2 * HID)), _full_spec((1, HID))],
    out_specs=[_row_spec(HID), _row_spec(HID)],
    out_shape=[jax.ShapeDtypeStruct((N, HID), jnp.float32)] * 2,
)

_tc2 = pl.pallas_call(
    _tc2_body,
    grid=(_GRID,),
    in_specs=[_row_spec(HID), _row_spec(HID), _row_spec(CNT_W), _row_spec(CNT_W),
              _row_spec(HID)],
    out_specs=[_row_spec(HID), _row_spec(CNT_W)],
    out_shape=[jax.ShapeDtypeStruct((N, HID), jnp.float32),
               jax.ShapeDtypeStruct((N, CNT_W), jnp.float32)],
)

_tc3 = pl.pallas_call(
    _tc3_body,
    grid=(_GRID,),
    in_specs=[_row_spec(HID), _row_spec(HID), _row_spec(CNT_W), _row_spec(HID),
              _full_spec((2 * HID, OUT)), _full_spec((1, OUT))],
    out_specs=_row_spec(OUT),
    out_shape=jax.ShapeDtypeStruct((N, OUT), jnp.float32),
)


def kernel(x, edge_index, W1_l, b1, W1_r, W2_l, b2, W2_r):
    w1 = jnp.concatenate([W1_l.T, W1_r.T], axis=1)
    w2 = jnp.concatenate([W2_l.T, W2_r.T], axis=0)
    edges_flat = edge_index.reshape(2 * E)
    y1, xr1 = _tc1(x, w1, b1[None, :])
    s1p, cntp = _seg_with_counts(y1, edges_flat)
    h, inv = _tc2(s1p[0, :N], s1p[1, :N], cntp[0, :N], cntp[1, :N], xr1)
    s2p = _seg_plain(h, edges_flat)
    out = _tc3(s2p[0, :N], s2p[1, :N], inv, h, w2, b2[None, :])
    return out
